# baseline (speedup 1.0000x reference)
import functools

import jax
import jax.numpy as jnp
from jax.experimental import pallas as pl
from jax.experimental.pallas import tpu as pltpu

N = 1024
G = 8
NB = 2
B = N // NB


def _dot(a, b):
    return jax.lax.dot_general(
        a, b, (((1,), (0,)), ((), ())),
        preferred_element_type=jnp.float32,
    )


def _dot_t(a, b):
    return jax.lax.dot_general(
        a, b, (((0,), (0,)), ((), ())),
        preferred_element_type=jnp.float32,
    )


def _gnn_body(x_hbm, batch_ref, w1_hbm, b1_ref, w2_hbm, b2_ref, out_ref,
              xs_ref, w1_ref, w2_ref, p1_ref, sem_x, sem_w):
    cw1 = pltpu.make_async_copy(w1_hbm, w1_ref, sem_w.at[0])
    cw1.start()
    chunk_copies = []
    for j in range(NB):
        c = pltpu.make_async_copy(
            x_hbm.at[pl.ds(j * B, B), :], xs_ref.at[pl.ds(j * B, B), :],
            sem_x.at[j])
        c.start()
        chunk_copies.append(c)
    cw2 = pltpu.make_async_copy(w2_hbm, w2_ref, sem_w.at[1])
    cw2.start()

    cw1.wait()
    w1 = w1_ref[...].astype(jnp.bfloat16)
    ones_b = jnp.ones((B, 1), jnp.bfloat16)
    deg_col = jnp.ones((N, 1), jnp.float32)
    xb16 = []
    for j in range(NB):
        chunk_copies[j].wait()
        x_b = xs_ref[pl.ds(j * B, B), :].astype(jnp.bfloat16)
        xb16.append(x_b)
        p1_ref[pl.ds(j * B, B), :] = _dot(x_b, w1)
        deg_col = deg_col + _dot_t(x_b, ones_b)
    x = jnp.concatenate(xb16, axis=0)
    s_col = jax.lax.rsqrt(deg_col)

    q1 = p1_ref[...] * s_col
    h1 = jax.nn.relu((_dot_t(x, q1.astype(jnp.bfloat16)) + q1) * s_col
                     + b1_ref[...])

    cw2.wait()
    q2 = _dot(h1.astype(jnp.bfloat16), w2_ref[...].astype(jnp.bfloat16)) * s_col
    seg = jax.lax.broadcasted_iota(jnp.int32, (N, G), 1)
    mt = (batch_ref[...] == seg).astype(jnp.float32)
    ones_col = jnp.ones((N, 1), jnp.bfloat16)
    cnt_col = _dot_t(mt.astype(jnp.bfloat16), ones_col)
    wt = mt * s_col
    vt = _dot(x, wt.astype(jnp.bfloat16)) + wt
    acc = (_dot_t(vt.astype(jnp.bfloat16), q2.astype(jnp.bfloat16))
           + cnt_col * b2_ref[...])
    out_ref[...] = acc / jnp.maximum(cnt_col, 1.0)


@functools.partial(jax.jit, static_argnames=())
def kernel(x, batch, W1, b1, W2, b2):
    return pl.pallas_call(
        _gnn_body,
        in_specs=[
            pl.BlockSpec(memory_space=pl.ANY),
            pl.BlockSpec(memory_space=pltpu.MemorySpace.VMEM),
            pl.BlockSpec(memory_space=pl.ANY),
            pl.BlockSpec(memory_space=pltpu.MemorySpace.VMEM),
            pl.BlockSpec(memory_space=pl.ANY),
            pl.BlockSpec(memory_space=pltpu.MemorySpace.VMEM),
        ],
        out_shape=jax.ShapeDtypeStruct((G, N), jnp.float32),
        scratch_shapes=[
            pltpu.VMEM((N, N), jnp.float32),
            pltpu.VMEM((N, N // 2), jnp.float32),
            pltpu.VMEM((N // 2, N), jnp.float32),
            pltpu.VMEM((N, N // 2), jnp.float32),
            pltpu.SemaphoreType.DMA((NB,)),
            pltpu.SemaphoreType.DMA((2,)),
        ],
    )(
        x.astype(jnp.float32),
        batch.astype(jnp.int32).reshape(N, 1),
        W1.astype(jnp.float32),
        b1.astype(jnp.float32).reshape(1, N // 2),
        W2.astype(jnp.float32),
        b2.astype(jnp.float32).reshape(1, N),
    )


# speedup vs baseline: 1.0252x; 1.0252x over previous
import functools

import jax
import jax.numpy as jnp
from jax.experimental import pallas as pl

N = 1024
G = 8


def _dot(a, b):
    return jax.lax.dot_general(
        a, b, (((1,), (0,)), ((), ())),
        preferred_element_type=jnp.float32,
        precision=jax.lax.Precision.DEFAULT,
    )


def _dot_t(a, b):
    return jax.lax.dot_general(
        a, b, (((0,), (0,)), ((), ())),
        preferred_element_type=jnp.float32,
        precision=jax.lax.Precision.DEFAULT,
    )


def _gnn_body(x_ref, batch_ref, w1_ref, b1_ref, w2_ref, b2_ref, out_ref):
    x = x_ref[...]

    ones_col = jnp.ones((N, 1), jnp.float32)
    deg_col = _dot_t(x, ones_col) + 1.0
    s_col = jax.lax.rsqrt(deg_col)

    q1 = _dot(x, w1_ref[...]) * s_col
    h1 = jax.nn.relu((_dot_t(x, q1) + q1) * s_col + b1_ref[...])

    q2 = _dot(h1, w2_ref[...]) * s_col
    seg = jax.lax.broadcasted_iota(jnp.int32, (N, G), 1)
    mt = (batch_ref[...] == seg).astype(jnp.float32)
    cnt_col = _dot_t(mt, ones_col)
    wt = mt * s_col
    vt = _dot(x, wt) + wt
    acc = _dot_t(vt, q2) + cnt_col * b2_ref[...]
    out_ref[...] = acc / jnp.maximum(cnt_col, 1.0)


@functools.partial(jax.jit, static_argnames=())
def kernel(x, batch, W1, b1, W2, b2):
    return pl.pallas_call(
        _gnn_body,
        out_shape=jax.ShapeDtypeStruct((G, N), jnp.float32),
    )(
        x.astype(jnp.float32),
        batch.astype(jnp.int32).reshape(N, 1),
        W1.astype(jnp.float32),
        b1.astype(jnp.float32).reshape(1, N // 2),
        W2.astype(jnp.float32),
        b2.astype(jnp.float32).reshape(1, N),
    )
